# baseline (speedup 1.0000x reference)
import functools

import jax
import jax.numpy as jnp
from jax import lax
from jax.experimental import pallas as pl
from jax.experimental.pallas import tpu as pltpu
from jax.experimental.pallas import tpu_sc as plsc

NUM_CORES = 2
NUM_SUBCORES = 16
NUM_WORKERS = NUM_CORES * NUM_SUBCORES
LANES = 16

D = 64
CHUNK = 128


def _sc_body(tok_hbm, ids_hbm, pos_hbm, out_hbm,
             idx_all, pos_v, buf0, buf1, sem0, sem1):
    n_chunks = ids_hbm.shape[0] // NUM_WORKERS
    wid = lax.axis_index("s") * NUM_CORES + lax.axis_index("c")
    idx_base = wid * n_chunks
    out_base = wid * n_chunks * CHUNK

    pltpu.sync_copy(ids_hbm.at[pl.ds(idx_base, n_chunks)], idx_all)
    pltpu.sync_copy(pos_hbm, pos_v)

    n_pos = pos_v.shape[0]

    pltpu.async_copy(tok_hbm.at[idx_all.at[0]], buf0, sem0)
    pltpu.async_copy(tok_hbm.at[idx_all.at[1]], buf1, sem1)

    def do_chunk(g, buf, sem):
        pltpu.make_async_copy(tok_hbm.at[pl.ds(0, CHUNK)], buf, sem).wait()

        base_l = lax.rem(g * CHUNK, n_pos)

        def radd(r, carry):
            m = base_l + r
            m = jnp.where(m >= n_pos, m - n_pos, m)
            for j in range(D // LANES):
                sl = pl.ds(j * LANES, LANES)
                plsc.addupdate(buf.at[r, sl], pos_v[m, sl])
            return carry

        lax.fori_loop(0, CHUNK, radd, 0)

        pltpu.sync_copy(buf, out_hbm.at[pl.ds(out_base + g * CHUNK, CHUNK)])

        @pl.when(g + 2 < n_chunks)
        def _():
            pltpu.async_copy(tok_hbm.at[idx_all.at[g + 2]], buf, sem)

    def pair(i, carry):
        do_chunk(2 * i, buf0, sem0)
        do_chunk(2 * i + 1, buf1, sem1)
        return carry

    lax.fori_loop(0, n_chunks // 2, pair, 0)


def _make_sc_kernel(n_flat):
    n_chunks_total = n_flat // CHUNK
    per_worker_chunks = n_chunks_total // NUM_WORKERS
    mesh = plsc.VectorSubcoreMesh(core_axis_name="c", subcore_axis_name="s")
    return pl.kernel(
        _sc_body,
        out_type=jax.ShapeDtypeStruct((n_flat, D), jnp.float32),
        mesh=mesh,
        scratch_types=[
            pltpu.VMEM((per_worker_chunks, CHUNK), jnp.int32),
            pltpu.VMEM((200, D), jnp.float32),
            pltpu.VMEM((CHUNK, D), jnp.float32),
            pltpu.VMEM((CHUNK, D), jnp.float32),
            pltpu.SemaphoreType.DMA,
            pltpu.SemaphoreType.DMA,
        ],
        compiler_params=pltpu.CompilerParams(use_tc_tiling_on_sc=False),
    )


@jax.jit
def kernel(ids, tok, pos):
    B, L = ids.shape
    n_flat = B * L
    ids2d = ids.astype(jnp.int32).reshape(n_flat // CHUNK, CHUNK)
    pos2d = pos.reshape(pos.shape[-2], pos.shape[-1]).astype(jnp.float32)
    out = _make_sc_kernel(n_flat)(tok, ids2d, pos2d)
    return out.reshape(B, L, D)


# speedup vs baseline: 1.0652x; 1.0652x over previous
import jax
import jax.numpy as jnp
from jax import lax
from jax.experimental import pallas as pl
from jax.experimental.pallas import tpu as pltpu
from jax.experimental.pallas import tpu_sc as plsc

NUM_CORES = 2
NUM_SUBCORES = 16
NUM_WORKERS = NUM_CORES * NUM_SUBCORES
LANES = 16

D = 64
CHUNK = 128
NBUF = 4
AHEAD = 2


def _sc_body(tok_hbm, ids_hbm, pos_hbm, out_hbm,
             idx_all, pos_v,
             buf0, buf1, buf2, buf3,
             g0, g1, g2, g3, s0, s1, s2, s3):
    bufs = (buf0, buf1, buf2, buf3)
    gsems = (g0, g1, g2, g3)
    ssems = (s0, s1, s2, s3)

    n_chunks = ids_hbm.shape[0] // NUM_WORKERS
    wid = lax.axis_index("s") * NUM_CORES + lax.axis_index("c")
    idx_base = wid * n_chunks
    out_base = wid * n_chunks * CHUNK

    pltpu.sync_copy(ids_hbm.at[pl.ds(idx_base, n_chunks)], idx_all)
    pltpu.sync_copy(pos_hbm, pos_v)

    n_pos = pos_v.shape[0]

    for k in range(AHEAD):
        pltpu.async_copy(tok_hbm.at[idx_all.at[k]], bufs[k], gsems[k])

    def process(g, k):
        buf, gsem, ssem = bufs[k], gsems[k], ssems[k]
        pltpu.make_async_copy(tok_hbm.at[pl.ds(0, CHUNK)], buf, gsem).wait()

        base_l = lax.rem(g * CHUNK, n_pos)

        def radd(r, carry):
            m = base_l + r
            m = jnp.where(m >= n_pos, m - n_pos, m)
            for j in range(D // LANES):
                sl = pl.ds(j * LANES, LANES)
                plsc.addupdate(buf.at[r, sl], pos_v[m, sl])
            return carry

        lax.fori_loop(0, CHUNK, radd, 0, unroll=4)

        pltpu.async_copy(buf, out_hbm.at[pl.ds(out_base + g * CHUNK, CHUNK)],
                         ssem)

        ka = (k + AHEAD) % NBUF
        ga = g + AHEAD

        @pl.when(ga < n_chunks)
        def _():
            @pl.when(ga >= NBUF)
            def _():
                pltpu.make_async_copy(
                    bufs[ka], out_hbm.at[pl.ds(0, CHUNK)], ssems[ka]).wait()
            pltpu.async_copy(tok_hbm.at[idx_all.at[ga]], bufs[ka], gsems[ka])

    def ring_lap(i, carry):
        g = NBUF * i
        for k in range(NBUF):
            process(g + k, k)
        return carry

    lax.fori_loop(0, n_chunks // NBUF, ring_lap, 0)

    for k in range(NBUF):
        pltpu.make_async_copy(bufs[k], out_hbm.at[pl.ds(0, CHUNK)],
                              ssems[k]).wait()


def _make_sc_kernel(n_flat):
    n_chunks_total = n_flat // CHUNK
    per_worker_chunks = n_chunks_total // NUM_WORKERS
    mesh = plsc.VectorSubcoreMesh(core_axis_name="c", subcore_axis_name="s")
    return pl.kernel(
        _sc_body,
        out_type=jax.ShapeDtypeStruct((n_flat, D), jnp.float32),
        mesh=mesh,
        scratch_types=[
            pltpu.VMEM((per_worker_chunks, CHUNK), jnp.int32),
            pltpu.VMEM((200, D), jnp.float32),
        ] + [pltpu.VMEM((CHUNK, D), jnp.float32)] * NBUF
          + [pltpu.SemaphoreType.DMA] * (2 * NBUF),
        compiler_params=pltpu.CompilerParams(use_tc_tiling_on_sc=False),
    )


@jax.jit
def kernel(ids, tok, pos):
    B, L = ids.shape
    n_flat = B * L
    ids2d = ids.astype(jnp.int32).reshape(n_flat // CHUNK, CHUNK)
    pos2d = pos.reshape(pos.shape[-2], pos.shape[-1]).astype(jnp.float32)
    out = _make_sc_kernel(n_flat)(tok, ids2d, pos2d)
    return out.reshape(B, L, D)


# speedup vs baseline: 1.2266x; 1.1514x over previous
import jax
import jax.numpy as jnp
from jax import lax
from jax.experimental import pallas as pl
from jax.experimental.pallas import tpu as pltpu
from jax.experimental.pallas import tpu_sc as plsc

NUM_CORES = 2
NUM_SUBCORES = 16
NUM_WORKERS = NUM_CORES * NUM_SUBCORES
LANES = 16

B, L, D = 4096, 200, 64
CHUNK = B // NUM_WORKERS
NBUF = 4
AHEAD = 2


def _sc_body(tok_hbm, ids_hbm, pos_hbm, out_hbm,
             idx_all, pos_v,
             buf0, buf1, buf2, buf3,
             g0, g1, g2, g3, s0, s1, s2, s3):
    bufs = (buf0, buf1, buf2, buf3)
    gsems = (g0, g1, g2, g3)
    ssems = (s0, s1, s2, s3)

    wid = lax.axis_index("s") * NUM_CORES + lax.axis_index("c")
    b0 = wid * CHUNK

    pltpu.sync_copy(ids_hbm.at[:, pl.ds(b0, CHUNK)], idx_all)
    pltpu.sync_copy(pos_hbm, pos_v)

    for k in range(AHEAD):
        pltpu.async_copy(tok_hbm.at[idx_all.at[k]], bufs[k], gsems[k])

    def process(l, k):
        buf, gsem, ssem = bufs[k], gsems[k], ssems[k]
        pltpu.make_async_copy(tok_hbm.at[pl.ds(0, CHUNK)], buf, gsem).wait()

        pvecs = [pos_v[l, pl.ds(j * LANES, LANES)] for j in range(D // LANES)]

        def radd(r, carry):
            for j in range(D // LANES):
                plsc.addupdate(buf.at[r, pl.ds(j * LANES, LANES)], pvecs[j])
            return carry

        lax.fori_loop(0, CHUNK, radd, 0, unroll=8)

        pltpu.async_copy(buf, out_hbm.at[pl.ds(b0, CHUNK), l], ssem)

        ka = (k + AHEAD) % NBUF
        la = l + AHEAD

        @pl.when(la < L)
        def _():
            @pl.when(la >= NBUF)
            def _():
                pltpu.make_async_copy(
                    bufs[ka], out_hbm.at[pl.ds(b0, CHUNK), 0],
                    ssems[ka]).wait()
            pltpu.async_copy(tok_hbm.at[idx_all.at[la]], bufs[ka], gsems[ka])

    def ring_lap(i, carry):
        l = NBUF * i
        for k in range(NBUF):
            process(l + k, k)
        return carry

    lax.fori_loop(0, L // NBUF, ring_lap, 0)

    for k in range(NBUF):
        pltpu.make_async_copy(bufs[k], out_hbm.at[pl.ds(b0, CHUNK), 0],
                              ssems[k]).wait()


def _make_sc_kernel():
    mesh = plsc.VectorSubcoreMesh(core_axis_name="c", subcore_axis_name="s")
    return pl.kernel(
        _sc_body,
        out_type=jax.ShapeDtypeStruct((B, L, D), jnp.float32),
        mesh=mesh,
        scratch_types=[
            pltpu.VMEM((L, CHUNK), jnp.int32),
            pltpu.VMEM((L, D), jnp.float32),
        ] + [pltpu.VMEM((CHUNK, D), jnp.float32)] * NBUF
          + [pltpu.SemaphoreType.DMA] * (2 * NBUF),
        compiler_params=pltpu.CompilerParams(use_tc_tiling_on_sc=False),
    )


def _kernel_impl(ids, tok, pos):
    ids_t = ids.astype(jnp.int32).T
    pos2d = pos.reshape(L, D).astype(jnp.float32)
    tok_a = lax.optimization_barrier(tok.reshape(500000, 128))
    tok_b = tok_a.reshape(1000000, 64)
    out = _make_sc_kernel()(tok_b, ids_t, pos2d)
    return out


kernel = jax.jit(_kernel_impl)


# speedup vs baseline: 1.6271x; 1.3266x over previous
import jax
import jax.numpy as jnp
from jax import lax
from jax.experimental import layout as jax_layout
from jax.experimental import pallas as pl
from jax.experimental.pallas import tpu as pltpu
from jax.experimental.pallas import tpu_sc as plsc

NUM_CORES = 2
NUM_SUBCORES = 16
NUM_WORKERS = NUM_CORES * NUM_SUBCORES
LANES = 16

B, L, D = 4096, 200, 64
CHUNK = B // NUM_WORKERS
NBUF = 4
AHEAD = 2


def _sc_body(tok_hbm, ids_hbm, pos_hbm, out_hbm,
             idx_all, pos_v,
             buf0, buf1, buf2, buf3,
             g0, g1, g2, g3, s0, s1, s2, s3):
    bufs = (buf0, buf1, buf2, buf3)
    gsems = (g0, g1, g2, g3)
    ssems = (s0, s1, s2, s3)

    wid = lax.axis_index("s") * NUM_CORES + lax.axis_index("c")
    b0 = wid * CHUNK

    pltpu.sync_copy(ids_hbm.at[:, pl.ds(b0, CHUNK)], idx_all)
    pltpu.sync_copy(pos_hbm, pos_v)

    for k in range(AHEAD):
        pltpu.async_copy(tok_hbm.at[idx_all.at[k]], bufs[k], gsems[k])

    def process(l, k):
        buf, gsem, ssem = bufs[k], gsems[k], ssems[k]
        pltpu.make_async_copy(tok_hbm.at[pl.ds(0, CHUNK)], buf, gsem).wait()

        pvecs = [pos_v[l, pl.ds(j * LANES, LANES)] for j in range(D // LANES)]

        def radd(r, carry):
            for j in range(D // LANES):
                plsc.addupdate(buf.at[r, pl.ds(j * LANES, LANES)], pvecs[j])
            return carry

        lax.fori_loop(0, CHUNK, radd, 0, unroll=8)

        pltpu.async_copy(buf, out_hbm.at[pl.ds(b0, CHUNK), l, pl.ds(0, D)],
                         ssem)

        ka = (k + AHEAD) % NBUF
        la = l + AHEAD

        @pl.when(la < L)
        def _():
            @pl.when(la >= NBUF)
            def _():
                pltpu.make_async_copy(
                    bufs[ka], out_hbm.at[pl.ds(b0, CHUNK), 0, pl.ds(0, D)],
                    ssems[ka]).wait()
            pltpu.async_copy(tok_hbm.at[idx_all.at[la]], bufs[ka], gsems[ka])

    def ring_lap(i, carry):
        l = NBUF * i
        for k in range(NBUF):
            process(l + k, k)
        return carry

    lax.fori_loop(0, L // NBUF, ring_lap, 0)

    for k in range(NBUF):
        pltpu.make_async_copy(bufs[k],
                              out_hbm.at[pl.ds(b0, CHUNK), 0, pl.ds(0, D)],
                              ssems[k]).wait()


def _make_sc_kernel():
    mesh = plsc.VectorSubcoreMesh(core_axis_name="c", subcore_axis_name="s")
    return pl.kernel(
        _sc_body,
        out_type=jax.ShapeDtypeStruct((B, L, 2 * D), jnp.float32),
        mesh=mesh,
        scratch_types=[
            pltpu.VMEM((L, CHUNK), jnp.int32),
            pltpu.VMEM((L, D), jnp.float32),
        ] + [pltpu.VMEM((CHUNK, D), jnp.float32)] * NBUF
          + [pltpu.SemaphoreType.DMA] * (2 * NBUF),
        compiler_params=pltpu.CompilerParams(use_tc_tiling_on_sc=False),
    )


def _kernel_impl(ids, tok, pos):
    ids_t = ids.astype(jnp.int32).T
    pos2d = pos.reshape(L, D).astype(jnp.float32)
    tok_a = lax.optimization_barrier(tok.reshape(500000, 128))
    tok_b = tok_a.reshape(1000000, 64)
    out_p = _make_sc_kernel()(tok_b, ids_t, pos2d)
    return out_p[:, :, :D]


kernel = jax.jit(_kernel_impl)


# speedup vs baseline: 1.6569x; 1.0183x over previous
import jax
import jax.numpy as jnp
from jax import lax
from jax.experimental import layout as jax_layout
from jax.experimental import pallas as pl
from jax.experimental.pallas import tpu as pltpu
from jax.experimental.pallas import tpu_sc as plsc

NUM_CORES = 2
NUM_SUBCORES = 16
NUM_WORKERS = NUM_CORES * NUM_SUBCORES
LANES = 16

B, L, D = 4096, 200, 64
CHUNK = B // NUM_WORKERS
NBUF = 5
AHEAD = 3


def _sc_body(tok_hbm, ids_hbm, pos_hbm, out_hbm,
             idx_all, pos_v,
             buf0, buf1, buf2, buf3, buf4,
             g0, g1, g2, g3, g4, s0, s1, s2, s3, s4):
    bufs = (buf0, buf1, buf2, buf3, buf4)
    gsems = (g0, g1, g2, g3, g4)
    ssems = (s0, s1, s2, s3, s4)

    wid = lax.axis_index("s") * NUM_CORES + lax.axis_index("c")
    b0 = wid * CHUNK

    pltpu.sync_copy(ids_hbm.at[:, pl.ds(b0, CHUNK)], idx_all)
    pltpu.sync_copy(pos_hbm, pos_v)

    for k in range(AHEAD):
        pltpu.async_copy(tok_hbm.at[idx_all.at[k]], bufs[k], gsems[k])

    def process(l, k):
        buf, gsem, ssem = bufs[k], gsems[k], ssems[k]
        pltpu.make_async_copy(tok_hbm.at[pl.ds(0, CHUNK)], buf, gsem).wait()

        pvecs = [pos_v[l, pl.ds(j * LANES, LANES)] for j in range(D // LANES)]

        def radd(r, carry):
            for j in range(D // LANES):
                plsc.addupdate(buf.at[r, pl.ds(j * LANES, LANES)], pvecs[j])
            return carry

        lax.fori_loop(0, CHUNK, radd, 0, unroll=8)

        pltpu.async_copy(buf, out_hbm.at[pl.ds(b0, CHUNK), l, pl.ds(0, D)],
                         ssem)

        ka = (k + AHEAD) % NBUF
        la = l + AHEAD

        @pl.when(la < L)
        def _():
            @pl.when(la >= NBUF)
            def _():
                pltpu.make_async_copy(
                    bufs[ka], out_hbm.at[pl.ds(b0, CHUNK), 0, pl.ds(0, D)],
                    ssems[ka]).wait()
            pltpu.async_copy(tok_hbm.at[idx_all.at[la]], bufs[ka], gsems[ka])

    def ring_lap(i, carry):
        l = NBUF * i
        for k in range(NBUF):
            process(l + k, k)
        return carry

    lax.fori_loop(0, L // NBUF, ring_lap, 0)

    for k in range(NBUF):
        pltpu.make_async_copy(bufs[k],
                              out_hbm.at[pl.ds(b0, CHUNK), 0, pl.ds(0, D)],
                              ssems[k]).wait()


def _make_sc_kernel():
    mesh = plsc.VectorSubcoreMesh(core_axis_name="c", subcore_axis_name="s")
    return pl.kernel(
        _sc_body,
        out_type=jax.ShapeDtypeStruct((B, L, 2 * D), jnp.float32),
        mesh=mesh,
        scratch_types=[
            pltpu.VMEM((L, CHUNK), jnp.int32),
            pltpu.VMEM((L, D), jnp.float32),
        ] + [pltpu.VMEM((CHUNK, D), jnp.float32)] * NBUF
          + [pltpu.SemaphoreType.DMA] * (2 * NBUF),
        compiler_params=pltpu.CompilerParams(use_tc_tiling_on_sc=False),
    )


def _kernel_impl(ids, tok, pos):
    ids_t = ids.astype(jnp.int32).T
    pos2d = pos.reshape(L, D).astype(jnp.float32)
    tok_a = lax.optimization_barrier(tok.reshape(500000, 128))
    tok_b = tok_a.reshape(1000000, 64)
    out_p = _make_sc_kernel()(tok_b, ids_t, pos2d)
    return out_p[:, :, :D]


kernel = jax.jit(_kernel_impl)
